# baseline (speedup 1.0000x reference)
import functools

import jax
import jax.numpy as jnp
from jax import lax
from jax.experimental import pallas as pl
from jax.experimental.pallas import tpu as pltpu
from jax.experimental.pallas import tpu_sc as plsc

NV = 40962
NF = 81920
NVC = 10242
B = 16
C = 32
O = 32
BC = B * C

NW = 32
NVCP = 10752
NVP = 41984

FB = 8
VB = 16

_FPW = NF // NW
_VPW = NVP // NW


_GDN = lax.GatherDimensionNumbers(
    offset_dims=(), collapsed_slice_dims=(0,), start_index_map=(0,))


def _splat(vec16, j):
    idx = jnp.full((16, 1), j, dtype=jnp.int32)
    return lax.gather(vec16, idx, _GDN, slice_sizes=(1,),
                      mode=lax.GatherScatterMode.PROMISE_IN_BOUNDS)


def _worker_id():
    return lax.axis_index("s") * 2 + lax.axis_index("c")


def _faces_body(x2p, gcols, wew, wns, out, idx_v, rows_v, wewv, wnsv, outv,
                sem, semi):
    wid = _worker_id()
    nblk = _FPW // FB

    def block(i, carry):
        f0 = wid * _FPW + i * FB
        pltpu.sync_copy(gcols.at[pl.ds(f0 * 9, FB * 9)], idx_v)
        pltpu.sync_copy(wew.at[pl.ds(f0 * 16, FB * 16)], wewv)
        pltpu.sync_copy(wns.at[pl.ds(f0 * 16, FB * 16)], wnsv)
        pltpu.async_copy(x2p.at[idx_v], rows_v, sem).wait()

        def face(f, c2):
            wrow_e = wewv[pl.ds(f * 16, 16)]
            wrow_n = wnsv[pl.ds(f * 16, 16)]
            for chunk in range(2):
                acc_e = [jnp.zeros((16,), jnp.float32) for _ in range(16)]
                acc_n = [jnp.zeros((16,), jnp.float32) for _ in range(16)]
                for j in range(9):
                    we = _splat(wrow_e, j)
                    wn = _splat(wrow_n, j)
                    for g in range(16):
                        v = rows_v[f * 9 + j, pl.ds(chunk * 256 + g * 16, 16)]
                        acc_e[g] = acc_e[g] + we * v
                        acc_n[g] = acc_n[g] + wn * v
                for g in range(16):
                    gg = chunk * 16 + g
                    off = (gg // 2) * 64 + (gg % 2) * 16
                    outv[f, pl.ds(off, 16)] = acc_e[g]
                    outv[f, pl.ds(off + 32, 16)] = acc_n[g]
            return c2

        lax.fori_loop(0, FB, face, 0)
        pltpu.sync_copy(outv, out.at[pl.ds(f0, FB)])
        return carry

    lax.fori_loop(0, nblk, block, 0)


def _faces_call(x2p, gcols, wew, wns):
    mesh = plsc.VectorSubcoreMesh(core_axis_name="c", subcore_axis_name="s")
    return pl.kernel(
        _faces_body,
        out_type=jax.ShapeDtypeStruct((NF, 2 * BC), jnp.float32),
        mesh=mesh,
        scratch_types=[
            pltpu.VMEM((FB * 9,), jnp.int32),
            pltpu.VMEM((FB * 9, BC), jnp.float32),
            pltpu.VMEM((FB * 16,), jnp.float32),
            pltpu.VMEM((FB * 16,), jnp.float32),
            pltpu.VMEM((FB, 2 * BC), jnp.float32),
            pltpu.SemaphoreType.DMA,
            pltpu.SemaphoreType.DMA,
        ],
    )(x2p, gcols, wew, wns)


def _mixa_kernel(x_ref, w0_ref, w1_ref, out_ref):
    x = x_ref[...]
    for bg in range(4):
        xs = x[:, bg * 128:(bg + 1) * 128]
        out_ref[0, :, bg * 128:(bg + 1) * 128] = jnp.dot(
            xs, w0_ref[...], preferred_element_type=jnp.float32)
        out_ref[1, :, bg * 128:(bg + 1) * 128] = jnp.dot(
            xs, w1_ref[...], preferred_element_type=jnp.float32)


def _mixb_kernel(g_ref, wg_ref, out_ref):
    g = g_ref[...]
    for bg in range(8):
        out_ref[:, bg * 64:(bg + 1) * 64] = jnp.dot(
            g[:, bg * 128:(bg + 1) * 128], wg_ref[...],
            preferred_element_type=jnp.float32)


def _verts_body(u01, gmix, ca, cb, w2, biasrow, out,
                idxa, idxb, rowsa, rowsb, wv, biasv, outv, sema, semb):
    wid = _worker_id()
    nblk = _VPW // VB
    pltpu.sync_copy(biasrow, biasv)

    def block(i, carry):
        v0 = wid * _VPW + i * VB
        pltpu.sync_copy(ca.at[pl.ds(v0 * 8, VB * 8)], idxa)
        pltpu.sync_copy(cb.at[pl.ds(v0 * 6, VB * 6)], idxb)
        pltpu.sync_copy(w2.at[pl.ds(v0 * 16, VB * 16)], wv)
        cpa = pltpu.async_copy(u01.at[idxa], rowsa, sema)
        cpb = pltpu.async_copy(gmix.at[idxb], rowsb, semb)
        cpa.wait()
        cpb.wait()

        def row(r, c2):
            wrow = wv[pl.ds(r * 16, 16)]
            for chunk in range(2):
                acc = [biasv[pl.ds(chunk * 256 + g * 16, 16)]
                       for g in range(16)]
                for j in range(8):
                    w = _splat(wrow, j)
                    for g in range(16):
                        v = rowsa[r * 8 + j,
                                  pl.ds(chunk * 256 + g * 16, 16)]
                        acc[g] = acc[g] + w * v
                for j in range(6):
                    w = _splat(wrow, 8 + j)
                    for g in range(16):
                        v = rowsb[r * 6 + j,
                                  pl.ds(chunk * 256 + g * 16, 16)]
                        acc[g] = acc[g] + w * v
                for g in range(16):
                    outv[r, pl.ds(chunk * 256 + g * 16, 16)] = acc[g]
            return c2

        lax.fori_loop(0, VB, row, 0)
        pltpu.sync_copy(outv, out.at[pl.ds(v0, VB)])
        return carry

    lax.fori_loop(0, nblk, block, 0)


def _verts_call(u01, gmix, ca, cb, w2, biasrow):
    mesh = plsc.VectorSubcoreMesh(core_axis_name="c", subcore_axis_name="s")
    return pl.kernel(
        _verts_body,
        out_type=jax.ShapeDtypeStruct((NVP, BC), jnp.float32),
        mesh=mesh,
        scratch_types=[
            pltpu.VMEM((VB * 8,), jnp.int32),
            pltpu.VMEM((VB * 6,), jnp.int32),
            pltpu.VMEM((VB * 8, BC), jnp.float32),
            pltpu.VMEM((VB * 6, BC), jnp.float32),
            pltpu.VMEM((VB * 16,), jnp.float32),
            pltpu.VMEM((BC,), jnp.float32),
            pltpu.VMEM((VB, BC), jnp.float32),
            pltpu.SemaphoreType.DMA,
            pltpu.SemaphoreType.DMA,
        ],
    )(u01, gmix, ca, cb, w2, biasrow)


def kernel(x_coarse, coeffs, bias, NS, EW,
           G_rows, G_cols, G_vals,
           L_rows, L_cols, L_vals,
           F2V_rows, F2V_cols, F2V_vals, v2p):
    f32 = jnp.float32
    i32 = jnp.int32

    x2 = jnp.transpose(x_coarse, (2, 0, 1)).reshape(NVC, BC)
    x2p = jnp.pad(x2, ((0, NVCP - NVC), (0, 0)))

    gc9 = jnp.transpose(G_cols.reshape(3, NF, 3), (1, 0, 2)).reshape(NF, 9)
    gv9 = jnp.transpose(G_vals.reshape(3, NF, 3), (1, 0, 2))
    wew9 = (EW[:, :, None] * gv9).reshape(NF, 9)
    wns9 = (NS[:, :, None] * gv9).reshape(NF, 9)
    gvalid = gc9 < NVC
    gcols = jnp.where(gvalid, gc9, 0).astype(i32).reshape(NF * 9)
    wew = jnp.pad(jnp.where(gvalid, wew9, 0.0), ((0, 0), (0, 7))).reshape(NF * 16)
    wns = jnp.pad(jnp.where(gvalid, wns9, 0.0), ((0, 0), (0, 7))).reshape(NF * 16)

    n_ids = jnp.arange(NVP, dtype=i32)
    lc7 = jnp.pad(L_cols.reshape(NV, 7), ((0, NVP - NV), (0, 0)))
    lv7 = jnp.pad(L_vals.reshape(NV, 7), ((0, NVP - NV), (0, 0)))
    lvalid = (lc7 < NVC) & (n_ids[:, None] < NV)
    idok = n_ids < NVC
    ca = jnp.concatenate(
        [jnp.where(idok, n_ids, 0)[:, None],
         jnp.where(lvalid, NVCP + lc7, 0)],
        axis=1).astype(i32).reshape(NVP * 8)
    fc6 = jnp.pad(F2V_cols.reshape(NV, 6), ((0, NVP - NV), (0, 0)))
    fv6 = jnp.pad(F2V_vals.reshape(NV, 6), ((0, NVP - NV), (0, 0)))
    cb = fc6.astype(i32).reshape(NVP * 6)
    w2 = jnp.concatenate(
        [idok.astype(f32)[:, None],
         jnp.where(lvalid, lv7, 0.0),
         fv6,
         jnp.zeros((NVP, 2), f32)],
        axis=1).reshape(NVP * 16)

    w0 = coeffs[:, :, 0].T.astype(f32)
    w1 = coeffs[:, :, 1].T.astype(f32)
    wg = jnp.concatenate([coeffs[:, :, 2].T, coeffs[:, :, 3].T], axis=0)
    eye4 = jnp.eye(4, dtype=f32)
    eye2 = jnp.eye(2, dtype=f32)
    w0k = jnp.kron(eye4, w0)
    w1k = jnp.kron(eye4, w1)
    wgk = jnp.kron(eye2, wg.astype(f32))
    biasrow = jnp.tile(bias.astype(f32), B)

    gface = _faces_call(x2p, gcols, wew, wns)

    u01 = pl.pallas_call(
        _mixa_kernel,
        grid=(NVCP // 512,),
        in_specs=[
            pl.BlockSpec((512, BC), lambda i: (i, 0)),
            pl.BlockSpec((128, 128), lambda i: (0, 0)),
            pl.BlockSpec((128, 128), lambda i: (0, 0)),
        ],
        out_specs=pl.BlockSpec((2, 512, BC), lambda i: (0, i, 0)),
        out_shape=jax.ShapeDtypeStruct((2, NVCP, BC), f32),
    )(x2p, w0k, w1k).reshape(2 * NVCP, BC)

    gmix = pl.pallas_call(
        _mixb_kernel,
        grid=(NF // 512,),
        in_specs=[
            pl.BlockSpec((512, 2 * BC), lambda i: (i, 0)),
            pl.BlockSpec((128, 64), lambda i: (0, 0)),
        ],
        out_specs=pl.BlockSpec((512, BC), lambda i: (i, 0)),
        out_shape=jax.ShapeDtypeStruct((NF, BC), f32),
    )(gface, wgk)

    outp = _verts_call(u01, gmix, ca, cb, w2, biasrow)

    out = outp[:NV].reshape(NV, B, O)
    return jnp.transpose(out, (1, 2, 0))


# speedup vs baseline: 6.5128x; 6.5128x over previous
import functools

import jax
import jax.numpy as jnp
from jax import lax
from jax.experimental import pallas as pl
from jax.experimental.pallas import tpu as pltpu
from jax.experimental.pallas import tpu_sc as plsc

NV = 40962
NF = 81920
NVC = 10242
B = 16
C = 32
O = 32
BC = B * C

NW = 32
NVCP = 10752
NVP = 41984

FB = 8
VB = 16

_FPW = NF // NW
_VPW = NVP // NW

_GDN = lax.GatherDimensionNumbers(
    offset_dims=(), collapsed_slice_dims=(0,), start_index_map=(0,))


def _splat(vec16, j):
    idx = jnp.full((16, 1), j, dtype=jnp.int32)
    return lax.gather(vec16, idx, _GDN, slice_sizes=(1,),
                      mode=lax.GatherScatterMode.PROMISE_IN_BOUNDS)


def _worker_id():
    return lax.axis_index("s") * 2 + lax.axis_index("c")


def _faces_body(x2p, gcols, wew, wns, out, idx_v, rows_v, wewv, wnsv, outv,
                sem, semi):
    wid = _worker_id()
    nblk = _FPW // FB

    def block(i, carry):
        f0 = wid * _FPW + i * FB
        pltpu.sync_copy(gcols.at[pl.ds(f0 * 9, FB * 9)], idx_v)
        pltpu.sync_copy(wew.at[pl.ds(f0 * 16, FB * 16)], wewv)
        pltpu.sync_copy(wns.at[pl.ds(f0 * 16, FB * 16)], wnsv)
        pltpu.async_copy(x2p.at[idx_v], rows_v, sem).wait()

        def face(f, c2):
            wrow_e = wewv[pl.ds(f * 16, 16)]
            wrow_n = wnsv[pl.ds(f * 16, 16)]
            for chunk in range(2):
                acc_e = [jnp.zeros((16,), jnp.float32) for _ in range(16)]
                acc_n = [jnp.zeros((16,), jnp.float32) for _ in range(16)]
                for j in range(9):
                    we = _splat(wrow_e, j)
                    wn = _splat(wrow_n, j)
                    for g in range(16):
                        col = chunk * 256 + g * 16
                        v = rows_v[f * 9 + j, col // 128, pl.ds(col % 128, 16)]
                        acc_e[g] = acc_e[g] + we * v
                        acc_n[g] = acc_n[g] + wn * v
                for g in range(16):
                    gg = chunk * 16 + g
                    offe = (gg // 2) * 64 + (gg % 2) * 16
                    offn = offe + 32
                    outv[f, offe // 128, pl.ds(offe % 128, 16)] = acc_e[g]
                    outv[f, offn // 128, pl.ds(offn % 128, 16)] = acc_n[g]
            return c2

        lax.fori_loop(0, FB, face, 0)
        pltpu.sync_copy(outv, out.at[pl.ds(f0, FB)])
        return carry

    lax.fori_loop(0, nblk, block, 0)


def _faces_call(x2p, gcols, wew, wns):
    mesh = plsc.VectorSubcoreMesh(core_axis_name="c", subcore_axis_name="s")
    return pl.kernel(
        _faces_body,
        out_type=jax.ShapeDtypeStruct((NF, 8, 128), jnp.float32),
        mesh=mesh,
        scratch_types=[
            pltpu.VMEM((FB * 9,), jnp.int32),
            pltpu.VMEM((FB * 9, 4, 128), jnp.float32),
            pltpu.VMEM((FB * 16,), jnp.float32),
            pltpu.VMEM((FB * 16,), jnp.float32),
            pltpu.VMEM((FB, 8, 128), jnp.float32),
            pltpu.SemaphoreType.DMA,
            pltpu.SemaphoreType.DMA,
        ],
    )(x2p, gcols, wew, wns)


def _mixa_kernel(x_ref, w0_ref, w1_ref, out_ref):
    x = x_ref[...]
    for bg in range(4):
        xs = x[:, bg * 128:(bg + 1) * 128]
        out_ref[0, :, bg * 128:(bg + 1) * 128] = jnp.dot(
            xs, w0_ref[...], preferred_element_type=jnp.float32)
        out_ref[1, :, bg * 128:(bg + 1) * 128] = jnp.dot(
            xs, w1_ref[...], preferred_element_type=jnp.float32)


def _mixb_kernel(g_ref, wg_ref, out_ref):
    g = g_ref[...]
    for bg in range(8):
        out_ref[:, bg * 64:(bg + 1) * 64] = jnp.dot(
            g[:, bg * 128:(bg + 1) * 128], wg_ref[...],
            preferred_element_type=jnp.float32)


def _verts_body(u01, gmix, ca, cb, w2, biasrow, out,
                idxa, idxb, rowsa, rowsb, wv, biasv, outv, sema, semb):
    wid = _worker_id()
    nblk = _VPW // VB
    pltpu.sync_copy(biasrow, biasv)

    def block(i, carry):
        v0 = wid * _VPW + i * VB
        pltpu.sync_copy(ca.at[pl.ds(v0 * 8, VB * 8)], idxa)
        pltpu.sync_copy(cb.at[pl.ds(v0 * 6, VB * 6)], idxb)
        pltpu.sync_copy(w2.at[pl.ds(v0 * 16, VB * 16)], wv)
        cpa = pltpu.async_copy(u01.at[idxa], rowsa, sema)
        cpb = pltpu.async_copy(gmix.at[idxb], rowsb, semb)
        cpa.wait()
        cpb.wait()

        def row(r, c2):
            wrow = wv[pl.ds(r * 16, 16)]
            for chunk in range(2):
                acc = [biasv[pl.ds(chunk * 256 + g * 16, 16)]
                       for g in range(16)]
                for j in range(8):
                    w = _splat(wrow, j)
                    for g in range(16):
                        col = chunk * 256 + g * 16
                        v = rowsa[r * 8 + j, col // 128, pl.ds(col % 128, 16)]
                        acc[g] = acc[g] + w * v
                for j in range(6):
                    w = _splat(wrow, 8 + j)
                    for g in range(16):
                        col = chunk * 256 + g * 16
                        v = rowsb[r * 6 + j, col // 128, pl.ds(col % 128, 16)]
                        acc[g] = acc[g] + w * v
                for g in range(16):
                    col = chunk * 256 + g * 16
                    outv[r, col // 128, pl.ds(col % 128, 16)] = acc[g]
            return c2

        lax.fori_loop(0, VB, row, 0)
        pltpu.sync_copy(outv, out.at[pl.ds(v0, VB)])
        return carry

    lax.fori_loop(0, nblk, block, 0)


def _verts_call(u01, gmix, ca, cb, w2, biasrow):
    mesh = plsc.VectorSubcoreMesh(core_axis_name="c", subcore_axis_name="s")
    return pl.kernel(
        _verts_body,
        out_type=jax.ShapeDtypeStruct((NVP, 4, 128), jnp.float32),
        mesh=mesh,
        scratch_types=[
            pltpu.VMEM((VB * 8,), jnp.int32),
            pltpu.VMEM((VB * 6,), jnp.int32),
            pltpu.VMEM((VB * 8, 4, 128), jnp.float32),
            pltpu.VMEM((VB * 6, 4, 128), jnp.float32),
            pltpu.VMEM((VB * 16,), jnp.float32),
            pltpu.VMEM((BC,), jnp.float32),
            pltpu.VMEM((VB, 4, 128), jnp.float32),
            pltpu.SemaphoreType.DMA,
            pltpu.SemaphoreType.DMA,
        ],
    )(u01, gmix, ca, cb, w2, biasrow)


def kernel(x_coarse, coeffs, bias, NS, EW,
           G_rows, G_cols, G_vals,
           L_rows, L_cols, L_vals,
           F2V_rows, F2V_cols, F2V_vals, v2p):
    f32 = jnp.float32
    i32 = jnp.int32

    x2 = jnp.transpose(x_coarse, (2, 0, 1)).reshape(NVC, BC)
    x2p = jnp.pad(x2, ((0, NVCP - NVC), (0, 0))).reshape(NVCP, 4, 128)

    gc9 = jnp.transpose(G_cols.reshape(3, NF, 3), (1, 0, 2)).reshape(NF, 9)
    gv9 = jnp.transpose(G_vals.reshape(3, NF, 3), (1, 0, 2))
    wew9 = (EW[:, :, None] * gv9).reshape(NF, 9)
    wns9 = (NS[:, :, None] * gv9).reshape(NF, 9)
    gvalid = gc9 < NVC
    gcols = jnp.where(gvalid, gc9, gc9 % NVC).astype(i32).reshape(NF * 9)
    wew = jnp.pad(jnp.where(gvalid, wew9, 0.0), ((0, 0), (0, 7))).reshape(NF * 16)
    wns = jnp.pad(jnp.where(gvalid, wns9, 0.0), ((0, 0), (0, 7))).reshape(NF * 16)

    n_ids = jnp.arange(NVP, dtype=i32)
    lc7 = jnp.pad(L_cols.reshape(NV, 7), ((0, NVP - NV), (0, 0)))
    lv7 = jnp.pad(L_vals.reshape(NV, 7), ((0, NVP - NV), (0, 0)))
    lvalid = (lc7 < NVC) & (n_ids[:, None] < NV)
    lspread = (lc7 + n_ids[:, None]) % NVC
    idok = n_ids < NVC
    ca = jnp.concatenate(
        [jnp.where(idok, n_ids, n_ids % NVC)[:, None],
         NVCP + jnp.where(lvalid, lc7, lspread)],
        axis=1).astype(i32).reshape(NVP * 8)
    fc6 = jnp.pad(F2V_cols.reshape(NV, 6), ((0, NVP - NV), (0, 0)))
    fv6 = jnp.pad(F2V_vals.reshape(NV, 6), ((0, NVP - NV), (0, 0)))
    fok = n_ids[:, None] < NV
    cb = jnp.where(fok, fc6,
                   (n_ids[:, None] + jnp.arange(6)[None, :]) % NF
                   ).astype(i32).reshape(NVP * 6)
    w2 = jnp.concatenate(
        [idok.astype(f32)[:, None],
         jnp.where(lvalid, lv7, 0.0),
         jnp.where(fok, fv6, 0.0),
         jnp.zeros((NVP, 2), f32)],
        axis=1).reshape(NVP * 16)

    w0 = coeffs[:, :, 0].T.astype(f32)
    w1 = coeffs[:, :, 1].T.astype(f32)
    wg = jnp.concatenate([coeffs[:, :, 2].T, coeffs[:, :, 3].T], axis=0)
    eye4 = jnp.eye(4, dtype=f32)
    eye2 = jnp.eye(2, dtype=f32)
    w0k = jnp.kron(eye4, w0)
    w1k = jnp.kron(eye4, w1)
    wgk = jnp.kron(eye2, wg.astype(f32))
    biasrow = jnp.tile(bias.astype(f32), B)

    gface = _faces_call(x2p.reshape(NVCP, 4, 128), gcols, wew, wns)

    u01 = pl.pallas_call(
        _mixa_kernel,
        grid=(NVCP // 512,),
        in_specs=[
            pl.BlockSpec((512, BC), lambda i: (i, 0)),
            pl.BlockSpec((128, 128), lambda i: (0, 0)),
            pl.BlockSpec((128, 128), lambda i: (0, 0)),
        ],
        out_specs=pl.BlockSpec((2, 512, BC), lambda i: (0, i, 0)),
        out_shape=jax.ShapeDtypeStruct((2, NVCP, BC), f32),
    )(x2p.reshape(NVCP, BC), w0k, w1k).reshape(2 * NVCP, 4, 128)

    gmix = pl.pallas_call(
        _mixb_kernel,
        grid=(NF // 512,),
        in_specs=[
            pl.BlockSpec((512, 2 * BC), lambda i: (i, 0)),
            pl.BlockSpec((128, 64), lambda i: (0, 0)),
        ],
        out_specs=pl.BlockSpec((512, BC), lambda i: (i, 0)),
        out_shape=jax.ShapeDtypeStruct((NF, BC), f32),
    )(gface.reshape(NF, 2 * BC), wgk).reshape(NF, 4, 128)

    outp = _verts_call(u01, gmix, ca, cb, w2, biasrow)

    out = outp.reshape(NVP, BC)[:NV].reshape(NV, B, O)
    return jnp.transpose(out, (1, 2, 0))


# speedup vs baseline: 12.1716x; 1.8689x over previous
import functools

import jax
import jax.numpy as jnp
from jax import lax
from jax.experimental import pallas as pl
from jax.experimental.pallas import tpu as pltpu
from jax.experimental.pallas import tpu_sc as plsc

NV = 40962
NF = 81920
NVC = 10242
B = 16
C = 32
O = 32
BC = B * C

NW = 32
NVCP = 10752
NVP = 41984

FB = 8
HF = FB // 2
VBB = 16
HV = VBB // 2

_FPW = NF // NW
_VPW = NVP // NW

_GDN = lax.GatherDimensionNumbers(
    offset_dims=(), collapsed_slice_dims=(0,), start_index_map=(0,))


def _splat(vec16, j):
    idx = jnp.full((16, 1), j, dtype=jnp.int32)
    return lax.gather(vec16, idx, _GDN, slice_sizes=(1,),
                      mode=lax.GatherScatterMode.PROMISE_IN_BOUNDS)


def _worker_id():
    return lax.axis_index("s") * 2 + lax.axis_index("c")


def _faces_body(x2p, gcols, warr, out, idxv, wv, rows, outv,
                si, sge, sgo, soe, soo):
    wid = _worker_id()
    nblk = _FPW // FB
    base = wid * _FPW

    def idx_issue(k):
        f0 = base + k * FB
        pltpu.async_copy(gcols.at[pl.ds(f0 * 9, FB * 9)], idxv.at[k % 2], si)
        pltpu.async_copy(warr.at[pl.ds(f0 * 32, FB * 32)], wv.at[k % 2], si)

    def idx_drain():
        pltpu.make_async_copy(gcols.at[pl.ds(0, FB * 9)], idxv.at[0], si).wait()
        pltpu.make_async_copy(warr.at[pl.ds(0, FB * 32)], wv.at[0], si).wait()

    def gather_issue(k, h, sem):
        pltpu.async_copy(
            x2p.at[idxv.at[k % 2, pl.ds(h * (HF * 9), HF * 9)]],
            rows.at[h], sem)

    def gather_drain(sem):
        pltpu.make_async_copy(
            x2p.at[idxv.at[0, pl.ds(0, HF * 9)]], rows.at[0], sem).wait()

    def store_drain(sem):
        pltpu.make_async_copy(outv.at[0], out.at[pl.ds(0, HF)], sem).wait()

    def compute_half(k, h, so_sem, guard):
        @pl.when(guard)
        def _():
            store_drain(so_sem)

        def face(f, c2):
            woff = (h * HF + f) * 32
            wrow_e = wv[k % 2, pl.ds(woff, 16)]
            wrow_n = wv[k % 2, pl.ds(woff + 16, 16)]
            for chunk in range(2):
                acc_e = [jnp.zeros((16,), jnp.float32) for _ in range(16)]
                acc_n = [jnp.zeros((16,), jnp.float32) for _ in range(16)]
                for j in range(9):
                    we = _splat(wrow_e, j)
                    wn = _splat(wrow_n, j)
                    for g in range(16):
                        col = chunk * 256 + g * 16
                        v = rows[h, f * 9 + j, col // 128,
                                 pl.ds(col % 128, 16)]
                        acc_e[g] = acc_e[g] + we * v
                        acc_n[g] = acc_n[g] + wn * v
                for g in range(16):
                    gg = chunk * 16 + g
                    offe = (gg // 2) * 64 + (gg % 2) * 16
                    offn = offe + 32
                    outv[h, f, offe // 128, pl.ds(offe % 128, 16)] = acc_e[g]
                    outv[h, f, offn // 128, pl.ds(offn % 128, 16)] = acc_n[g]
            return c2

        lax.fori_loop(0, HF, face, 0)
        f0 = base + k * FB + h * HF
        pltpu.async_copy(outv.at[h], out.at[pl.ds(f0, HF)], so_sem)

    pltpu.sync_copy(gcols.at[pl.ds(base * 9, FB * 9)], idxv.at[0])
    pltpu.sync_copy(warr.at[pl.ds(base * 32, FB * 32)], wv.at[0])
    pltpu.async_copy(x2p.at[idxv.at[0, pl.ds(0, HF * 9)]], rows.at[0], sge)
    idx_issue(1)

    def block(k, carry):
        gather_issue(k, 1, sgo)
        gather_drain(sge)
        compute_half(k, 0, soe, k >= 1)

        @pl.when(k + 1 < nblk)
        def _():
            idx_drain()
            gather_issue(k + 1, 0, sge)

        gather_drain(sgo)
        compute_half(k, 1, soo, k >= 1)

        @pl.when(k + 2 < nblk)
        def _():
            idx_issue(k + 2)
        return carry

    lax.fori_loop(0, nblk, block, 0)
    store_drain(soe)
    store_drain(soo)


def _faces_call(x2p, gcols, warr):
    mesh = plsc.VectorSubcoreMesh(core_axis_name="c", subcore_axis_name="s")
    return pl.kernel(
        _faces_body,
        out_type=jax.ShapeDtypeStruct((NF, 8, 128), jnp.float32),
        mesh=mesh,
        scratch_types=[
            pltpu.VMEM((2, FB * 9), jnp.int32),
            pltpu.VMEM((2, FB * 32), jnp.float32),
            pltpu.VMEM((2, HF * 9, 4, 128), jnp.float32),
            pltpu.VMEM((2, HF, 8, 128), jnp.float32),
            pltpu.SemaphoreType.DMA,
            pltpu.SemaphoreType.DMA,
            pltpu.SemaphoreType.DMA,
            pltpu.SemaphoreType.DMA,
            pltpu.SemaphoreType.DMA,
        ],
    )(x2p, gcols, warr)


def _mixa_kernel(x_ref, w0_ref, w1_ref, out_ref):
    x = x_ref[...]
    for bg in range(4):
        xs = x[:, bg * 128:(bg + 1) * 128]
        out_ref[0, :, bg * 128:(bg + 1) * 128] = jnp.dot(
            xs, w0_ref[...], preferred_element_type=jnp.float32)
        out_ref[1, :, bg * 128:(bg + 1) * 128] = jnp.dot(
            xs, w1_ref[...], preferred_element_type=jnp.float32)


def _mixb_kernel(g_ref, wg_ref, out_ref):
    g = g_ref[...]
    for bg in range(8):
        out_ref[:, bg * 64:(bg + 1) * 64] = jnp.dot(
            g[:, bg * 128:(bg + 1) * 128], wg_ref[...],
            preferred_element_type=jnp.float32)


def _verts_body(u01, gmix, ca, cb, w2, biasrow, out,
                idxa, idxb, wv, biasv, rowsa, rowsb, outv,
                si, sae, sao, sbe, sbo, soe, soo):
    wid = _worker_id()
    nblk = _VPW // VBB
    base = wid * _VPW
    pltpu.sync_copy(biasrow, biasv)

    def idx_issue(k):
        v0 = base + k * VBB
        pltpu.async_copy(ca.at[pl.ds(v0 * 8, VBB * 8)], idxa.at[k % 2], si)
        pltpu.async_copy(cb.at[pl.ds(v0 * 6, VBB * 6)], idxb.at[k % 2], si)
        pltpu.async_copy(w2.at[pl.ds(v0 * 16, VBB * 16)], wv.at[k % 2], si)

    def idx_drain():
        pltpu.make_async_copy(ca.at[pl.ds(0, VBB * 8)], idxa.at[0], si).wait()
        pltpu.make_async_copy(cb.at[pl.ds(0, VBB * 6)], idxb.at[0], si).wait()
        pltpu.make_async_copy(w2.at[pl.ds(0, VBB * 16)], wv.at[0], si).wait()

    def gather_issue(k, h, sa, sb):
        pltpu.async_copy(
            u01.at[idxa.at[k % 2, pl.ds(h * (HV * 8), HV * 8)]],
            rowsa.at[h], sa)
        pltpu.async_copy(
            gmix.at[idxb.at[k % 2, pl.ds(h * (HV * 6), HV * 6)]],
            rowsb.at[h], sb)

    def gather_drain(sa, sb):
        pltpu.make_async_copy(
            u01.at[idxa.at[0, pl.ds(0, HV * 8)]], rowsa.at[0], sa).wait()
        pltpu.make_async_copy(
            gmix.at[idxb.at[0, pl.ds(0, HV * 6)]], rowsb.at[0], sb).wait()

    def store_drain(sem):
        pltpu.make_async_copy(outv.at[0], out.at[pl.ds(0, HV)], sem).wait()

    def compute_half(k, h, so_sem, guard):
        @pl.when(guard)
        def _():
            store_drain(so_sem)

        def row(r, c2):
            woff = (h * HV + r) * 16
            wrow = wv[k % 2, pl.ds(woff, 16)]
            for chunk in range(2):
                acc = [biasv[pl.ds(chunk * 256 + g * 16, 16)]
                       for g in range(16)]
                for j in range(8):
                    w = _splat(wrow, j)
                    for g in range(16):
                        col = chunk * 256 + g * 16
                        v = rowsa[h, r * 8 + j, col // 128,
                                  pl.ds(col % 128, 16)]
                        acc[g] = acc[g] + w * v
                for j in range(6):
                    w = _splat(wrow, 8 + j)
                    for g in range(16):
                        col = chunk * 256 + g * 16
                        v = rowsb[h, r * 6 + j, col // 128,
                                  pl.ds(col % 128, 16)]
                        acc[g] = acc[g] + w * v
                for g in range(16):
                    col = chunk * 256 + g * 16
                    outv[h, r, col // 128, pl.ds(col % 128, 16)] = acc[g]
            return c2

        lax.fori_loop(0, HV, row, 0)
        v0 = base + k * VBB + h * HV
        pltpu.async_copy(outv.at[h], out.at[pl.ds(v0, HV)], so_sem)

    pltpu.sync_copy(ca.at[pl.ds(base * 8, VBB * 8)], idxa.at[0])
    pltpu.sync_copy(cb.at[pl.ds(base * 6, VBB * 6)], idxb.at[0])
    pltpu.sync_copy(w2.at[pl.ds(base * 16, VBB * 16)], wv.at[0])
    pltpu.async_copy(u01.at[idxa.at[0, pl.ds(0, HV * 8)]], rowsa.at[0], sae)
    pltpu.async_copy(gmix.at[idxb.at[0, pl.ds(0, HV * 6)]], rowsb.at[0], sbe)
    idx_issue(1)

    def block(k, carry):
        gather_issue(k, 1, sao, sbo)
        gather_drain(sae, sbe)
        compute_half(k, 0, soe, k >= 1)

        @pl.when(k + 1 < nblk)
        def _():
            idx_drain()
            gather_issue(k + 1, 0, sae, sbe)

        gather_drain(sao, sbo)
        compute_half(k, 1, soo, k >= 1)

        @pl.when(k + 2 < nblk)
        def _():
            idx_issue(k + 2)
        return carry

    lax.fori_loop(0, nblk, block, 0)
    store_drain(soe)
    store_drain(soo)


def _verts_call(u01, gmix, ca, cb, w2, biasrow):
    mesh = plsc.VectorSubcoreMesh(core_axis_name="c", subcore_axis_name="s")
    return pl.kernel(
        _verts_body,
        out_type=jax.ShapeDtypeStruct((NVP, 4, 128), jnp.float32),
        mesh=mesh,
        scratch_types=[
            pltpu.VMEM((2, VBB * 8), jnp.int32),
            pltpu.VMEM((2, VBB * 6), jnp.int32),
            pltpu.VMEM((2, VBB * 16), jnp.float32),
            pltpu.VMEM((BC,), jnp.float32),
            pltpu.VMEM((2, HV * 8, 4, 128), jnp.float32),
            pltpu.VMEM((2, HV * 6, 4, 128), jnp.float32),
            pltpu.VMEM((2, HV, 4, 128), jnp.float32),
            pltpu.SemaphoreType.DMA,
            pltpu.SemaphoreType.DMA,
            pltpu.SemaphoreType.DMA,
            pltpu.SemaphoreType.DMA,
            pltpu.SemaphoreType.DMA,
            pltpu.SemaphoreType.DMA,
            pltpu.SemaphoreType.DMA,
        ],
    )(u01, gmix, ca, cb, w2, biasrow)


def kernel(x_coarse, coeffs, bias, NS, EW,
           G_rows, G_cols, G_vals,
           L_rows, L_cols, L_vals,
           F2V_rows, F2V_cols, F2V_vals, v2p):
    f32 = jnp.float32
    i32 = jnp.int32

    x2 = jnp.transpose(x_coarse, (2, 0, 1)).reshape(NVC, BC)
    x2p = jnp.pad(x2, ((0, NVCP - NVC), (0, 0)))

    gc9 = jnp.transpose(G_cols.reshape(3, NF, 3), (1, 0, 2)).reshape(NF, 9)
    gv9 = jnp.transpose(G_vals.reshape(3, NF, 3), (1, 0, 2))
    wew9 = (EW[:, :, None] * gv9).reshape(NF, 9)
    wns9 = (NS[:, :, None] * gv9).reshape(NF, 9)
    gvalid = gc9 < NVC
    gcols = jnp.where(gvalid, gc9, gc9 % NVC).astype(i32).reshape(NF * 9)
    warr = jnp.concatenate(
        [jnp.pad(jnp.where(gvalid, wew9, 0.0), ((0, 0), (0, 7))),
         jnp.pad(jnp.where(gvalid, wns9, 0.0), ((0, 0), (0, 7)))],
        axis=1).reshape(NF * 32)

    n_ids = jnp.arange(NVP, dtype=i32)
    lc7 = jnp.pad(L_cols.reshape(NV, 7), ((0, NVP - NV), (0, 0)))
    lv7 = jnp.pad(L_vals.reshape(NV, 7), ((0, NVP - NV), (0, 0)))
    lvalid = (lc7 < NVC) & (n_ids[:, None] < NV)
    lspread = (lc7 + n_ids[:, None]) % NVC
    idok = n_ids < NVC
    ca = jnp.concatenate(
        [jnp.where(idok, n_ids, n_ids % NVC)[:, None],
         NVCP + jnp.where(lvalid, lc7, lspread)],
        axis=1).astype(i32).reshape(NVP * 8)
    fc6 = jnp.pad(F2V_cols.reshape(NV, 6), ((0, NVP - NV), (0, 0)))
    fv6 = jnp.pad(F2V_vals.reshape(NV, 6), ((0, NVP - NV), (0, 0)))
    fok = n_ids[:, None] < NV
    cb = jnp.where(fok, fc6,
                   (n_ids[:, None] + jnp.arange(6)[None, :]) % NF
                   ).astype(i32).reshape(NVP * 6)
    w2 = jnp.concatenate(
        [idok.astype(f32)[:, None],
         jnp.where(lvalid, lv7, 0.0),
         jnp.where(fok, fv6, 0.0),
         jnp.zeros((NVP, 2), f32)],
        axis=1).reshape(NVP * 16)

    w0 = coeffs[:, :, 0].T.astype(f32)
    w1 = coeffs[:, :, 1].T.astype(f32)
    wg = jnp.concatenate([coeffs[:, :, 2].T, coeffs[:, :, 3].T], axis=0)
    eye4 = jnp.eye(4, dtype=f32)
    eye2 = jnp.eye(2, dtype=f32)
    w0k = jnp.kron(eye4, w0)
    w1k = jnp.kron(eye4, w1)
    wgk = jnp.kron(eye2, wg.astype(f32))
    biasrow = jnp.tile(bias.astype(f32), B)

    gface = _faces_call(x2p.reshape(NVCP, 4, 128), gcols, warr)

    u01 = pl.pallas_call(
        _mixa_kernel,
        grid=(NVCP // 512,),
        in_specs=[
            pl.BlockSpec((512, BC), lambda i: (i, 0)),
            pl.BlockSpec((128, 128), lambda i: (0, 0)),
            pl.BlockSpec((128, 128), lambda i: (0, 0)),
        ],
        out_specs=pl.BlockSpec((2, 512, BC), lambda i: (0, i, 0)),
        out_shape=jax.ShapeDtypeStruct((2, NVCP, BC), f32),
    )(x2p, w0k, w1k).reshape(2 * NVCP, 4, 128)

    gmix = pl.pallas_call(
        _mixb_kernel,
        grid=(NF // 512,),
        in_specs=[
            pl.BlockSpec((512, 2 * BC), lambda i: (i, 0)),
            pl.BlockSpec((128, 64), lambda i: (0, 0)),
        ],
        out_specs=pl.BlockSpec((512, BC), lambda i: (i, 0)),
        out_shape=jax.ShapeDtypeStruct((NF, BC), f32),
    )(gface.reshape(NF, 2 * BC), wgk).reshape(NF, 4, 128)

    outp = _verts_call(u01, gmix, ca, cb, w2, biasrow)

    out = outp.reshape(NVP, BC)[:NV].reshape(NV, B, O)
    return jnp.transpose(out, (1, 2, 0))
